# baseline (speedup 1.0000x reference)
import jax
import jax.numpy as jnp
from jax import lax
from jax.experimental import pallas as pl
from jax.experimental.pallas import tpu as pltpu
from jax.experimental.pallas import tpu_sc as plsc

BATCH = 16384
FACTORS = 64
L = 16
NC, NS = 2, 16
NW = NC * NS
BPW = BATCH // NW
NG = BPW // L
GU = 16
NUSERS = 1000000
NTILES = NUSERS // GU
BLK = 128
NBLK = NUSERS // BLK
TAILBASE = NBLK * BLK
BPWK = NBLK // NW + 1


def _retile_body(ut_r, it_r, su_r, si_r, vin, vout, semin, semout):
    wid = lax.axis_index("s") * NC + lax.axis_index("c")
    lanes = lax.iota(jnp.int32, L)
    fvecs = [c * L + lanes for c in range(FACTORS // L)]

    def shuffle(p):
        def gt_step(gt, carry):
            base = gt * GU
            for r in range(8):
                for h in range(2):
                    tvec = jnp.full((L,), base + 2 * r + h, jnp.int32)
                    for c in range(FACTORS // L):
                        v = plsc.load_gather(vin.at[p], [fvecs[c], tvec])
                        vout[p, gt, r, pl.ds(FACTORS * h + L * c, L)] = v
            return carry

        lax.fori_loop(0, 8, gt_step, 0)

    for t_r, o_r in ((ut_r, su_r), (it_r, si_r)):
        def blk_of(b):
            return jnp.minimum(wid + NW * b, NBLK - 1)

        def fire_in(p, cb):
            pltpu.async_copy(
                t_r.at[:, pl.ds(cb * BLK, BLK)], vin.at[p], semin)

        def drain_in(p):
            pltpu.make_async_copy(
                t_r.at[:, pl.ds(0, BLK)], vin.at[p], semin).wait()

        def fire_out(p, cb):
            pltpu.async_copy(
                vout.at[p], o_r.at[pl.ds(cb * 8, 8)], semout)

        def drain_out(p):
            pltpu.make_async_copy(
                vout.at[p], o_r.at[pl.ds(0, 8)], semout).wait()

        fire_in(0, blk_of(0))
        fire_in(1, blk_of(1))
        for p in range(2):
            drain_in(p)
            shuffle(p)
            fire_out(p, blk_of(p))
            fire_in(p, blk_of(p + 2))

        def pair(tt, carry):
            for p in range(2):
                b = 2 * tt + p
                drain_in(p)
                drain_out(p)
                shuffle(p)
                fire_out(p, blk_of(b))
                fire_in(p, blk_of(b + 2))
            return carry

        lax.fori_loop(1, BPWK // 2 + 1, pair, 0)
        for p in range(2):
            drain_in(p)
            drain_out(p)


def _gather_body(user_r, item_i_r, item_j_r, su_r, si_r, tu_r, ti_r, out_r,
                 idx_u, idx_i, idx_j, bu, bi, bj, tuv, tiv, tbuf, out_v, sem):
    wid = lax.axis_index("s") * NC + lax.axis_index("c")

    pltpu.sync_copy(user_r.at[wid], idx_u)
    pltpu.sync_copy(item_i_r.at[wid], idx_i)
    pltpu.sync_copy(item_j_r.at[wid], idx_j)
    pltpu.sync_copy(tu_r, tuv)
    pltpu.sync_copy(ti_r, tiv)

    lanes = lax.iota(jnp.int32, L)
    trow = [(c * L + lanes) // 2 for c in range(FACTORS // L)]
    tcol = [FACTORS * ((c * L + lanes) % 2) for c in range(FACTORS // L)]

    def group(g, carry):
        kuv = idx_u[pl.ds(g * L, L)]
        kiv = idx_i[pl.ds(g * L, L)]
        kjv = idx_j[pl.ds(g * L, L)]
        guv = lax.shift_right_logical(kuv, 4)
        giv = lax.shift_right_logical(kiv, 4)
        gjv = lax.shift_right_logical(kjv, 4)
        suv = lax.bitwise_and(kuv, jnp.int32(GU - 1))
        siv = lax.bitwise_and(kiv, jnp.int32(GU - 1))
        sjv = lax.bitwise_and(kjv, jnp.int32(GU - 1))
        cu = pltpu.async_copy(su_r.at[guv], bu, sem)
        ci = pltpu.async_copy(si_r.at[giv], bi, sem)
        cj = pltpu.async_copy(si_r.at[gjv], bj, sem)
        cu.wait()
        ci.wait()
        cj.wait()
        for s in range(L):
            ku = kuv[s]
            ki = kiv[s]
            kj = kjv[s]
            su = suv[s]
            si = siv[s]
            sj = sjv[s]
            ru_ = lax.shift_right_logical(su, 1)
            ri_ = lax.shift_right_logical(si, 1)
            rj_ = lax.shift_right_logical(sj, 1)
            hu = lax.bitwise_and(su, jnp.int32(1)) * FACTORS
            hi = lax.bitwise_and(si, jnp.int32(1)) * FACTORS
            hj = lax.bitwise_and(sj, jnp.int32(1)) * FACTORS
            acc = jnp.zeros((L,), jnp.float32)
            for c in range(FACTORS // L):
                u = bu[s, ru_, pl.ds(hu + c * L, L)]
                vi = bi[s, ri_, pl.ds(hi + c * L, L)]
                vj = bj[s, rj_, pl.ds(hj + c * L, L)]
                acc = acc + u * (vi - vj)
            plsc.store_scatter(
                tbuf, [lanes, jnp.full((L,), s, jnp.int32)], acc)
            tl_u = ku >= TAILBASE
            tl_i = ki >= TAILBASE
            tl_j = kj >= TAILBASE

            @pl.when(tl_u | tl_i | tl_j)
            def _fix():
                tu_ = jnp.maximum(ku - TAILBASE, 0)
                ti_ = jnp.maximum(ki - TAILBASE, 0)
                tj_ = jnp.maximum(kj - TAILBASE, 0)
                acc2 = jnp.zeros((L,), jnp.float32)
                for c in range(FACTORS // L):
                    u = jnp.where(
                        tl_u, plsc.load_gather(tuv, [trow[c], tcol[c] + tu_]),
                        bu[s, ru_, pl.ds(hu + c * L, L)])
                    vi = jnp.where(
                        tl_i, plsc.load_gather(tiv, [trow[c], tcol[c] + ti_]),
                        bi[s, ri_, pl.ds(hi + c * L, L)])
                    vj = jnp.where(
                        tl_j, plsc.load_gather(tiv, [trow[c], tcol[c] + tj_]),
                        bj[s, rj_, pl.ds(hj + c * L, L)])
                    acc2 = acc2 + u * (vi - vj)
                plsc.store_scatter(
                    tbuf, [lanes, jnp.full((L,), s, jnp.int32)], acc2)
        tot = tbuf[0, :]
        for r in range(1, L):
            tot = tot + tbuf[r, :]
        out_v[pl.ds(g * L, L)] = tot
        return carry

    lax.fori_loop(0, NG, group, 0)
    pltpu.sync_copy(out_v, out_r.at[pl.ds(wid * BPW, BPW)])


def kernel(user, item_i, item_j, user_factors, item_factors):
    user2 = user.reshape(NW, BPW)
    ii2 = item_i.reshape(NW, BPW)
    ij2 = item_j.reshape(NW, BPW)
    ut = user_factors.T
    it = item_factors.T
    tu2 = lax.slice(ut, (0, TAILBASE), (FACTORS, NUSERS)).reshape(32, 128)
    ti2 = lax.slice(it, (0, TAILBASE), (FACTORS, NUSERS)).reshape(32, 128)
    mesh = plsc.VectorSubcoreMesh(core_axis_name="c", subcore_axis_name="s")
    params = pltpu.CompilerParams(needs_layout_passes=False)

    retile = pl.kernel(
        _retile_body,
        out_type=(
            jax.ShapeDtypeStruct((NTILES, 8, 2 * FACTORS), jnp.float32),
            jax.ShapeDtypeStruct((NTILES, 8, 2 * FACTORS), jnp.float32),
        ),
        mesh=mesh,
        compiler_params=params,
        scratch_types=[
            pltpu.VMEM((2, FACTORS, BLK), jnp.float32),
            pltpu.VMEM((2, 8, 8, 2 * FACTORS), jnp.float32),
            pltpu.SemaphoreType.DMA,
            pltpu.SemaphoreType.DMA,
        ],
    )
    su3, si3 = retile(ut, it)

    gather = pl.kernel(
        _gather_body,
        out_type=jax.ShapeDtypeStruct((BATCH,), jnp.float32),
        mesh=mesh,
        compiler_params=params,
        scratch_types=[
            pltpu.VMEM((BPW,), jnp.int32),
            pltpu.VMEM((BPW,), jnp.int32),
            pltpu.VMEM((BPW,), jnp.int32),
            pltpu.VMEM((L, 8, 2 * FACTORS), jnp.float32),
            pltpu.VMEM((L, 8, 2 * FACTORS), jnp.float32),
            pltpu.VMEM((L, 8, 2 * FACTORS), jnp.float32),
            pltpu.VMEM((32, 128), jnp.float32),
            pltpu.VMEM((32, 128), jnp.float32),
            pltpu.VMEM((L, L), jnp.float32),
            pltpu.VMEM((BPW,), jnp.float32),
            pltpu.SemaphoreType.DMA,
        ],
    )
    return gather(user2, ii2, ij2, su3, si3, tu2, ti2)


# speedup vs baseline: 2.7623x; 2.7623x over previous
import jax
import jax.numpy as jnp
from jax import lax
from jax.experimental import pallas as pl
from jax.experimental.pallas import tpu as pltpu
from jax.experimental.pallas import tpu_sc as plsc

BATCH = 16384
FACTORS = 64
L = 16
NC, NS = 2, 16
NW = NC * NS
BPW = BATCH // NW
NG = BPW // L


def _body(user_r, item_i_r, item_j_r, uf_r, if_r, out_r,
          idx_u, idx_i, idx_j, bu, bi, bj, tbuf, out_v, sem):
    wid = lax.axis_index("s") * NC + lax.axis_index("c")

    pltpu.sync_copy(user_r.at[wid], idx_u)
    pltpu.sync_copy(item_i_r.at[wid], idx_i)
    pltpu.sync_copy(item_j_r.at[wid], idx_j)

    lanes = lax.iota(jnp.int32, L)

    def group(g, carry):
        kuv = idx_u[pl.ds(g * L, L)]
        kiv = idx_i[pl.ds(g * L, L)]
        kjv = idx_j[pl.ds(g * L, L)]
        cu = pltpu.async_copy(
            uf_r.at[lax.shift_right_logical(kuv, 1)], bu, sem)
        ci = pltpu.async_copy(
            if_r.at[lax.shift_right_logical(kiv, 1)], bi, sem)
        cj = pltpu.async_copy(
            if_r.at[lax.shift_right_logical(kjv, 1)], bj, sem)
        huv = lax.bitwise_and(kuv, jnp.int32(1)) * FACTORS
        hiv = lax.bitwise_and(kiv, jnp.int32(1)) * FACTORS
        hjv = lax.bitwise_and(kjv, jnp.int32(1)) * FACTORS
        cu.wait()
        ci.wait()
        cj.wait()
        for s in range(L):
            hu = huv[s]
            hi = hiv[s]
            hj = hjv[s]
            acc = jnp.zeros((L,), jnp.float32)
            for c in range(FACTORS // L):
                u = bu[s, pl.ds(hu + c * L, L)]
                vi = bi[s, pl.ds(hi + c * L, L)]
                vj = bj[s, pl.ds(hj + c * L, L)]
                acc = acc + u * (vi - vj)
            plsc.store_scatter(
                tbuf, [lanes, jnp.full((L,), s, jnp.int32)], acc)
        tot = tbuf[0, :]
        for r in range(1, L):
            tot = tot + tbuf[r, :]
        out_v[pl.ds(g * L, L)] = tot
        return carry

    lax.fori_loop(0, NG, group, 0)
    pltpu.sync_copy(out_v, out_r.at[pl.ds(wid * BPW, BPW)])


def kernel(user, item_i, item_j, user_factors, item_factors):
    user2 = user.reshape(NW, BPW)
    ii2 = item_i.reshape(NW, BPW)
    ij2 = item_j.reshape(NW, BPW)
    mesh = plsc.VectorSubcoreMesh(core_axis_name="c", subcore_axis_name="s")
    params = pltpu.CompilerParams(needs_layout_passes=False)

    gather = pl.kernel(
        _body,
        out_type=jax.ShapeDtypeStruct((BATCH,), jnp.float32),
        mesh=mesh,
        compiler_params=params,
        scratch_types=[
            pltpu.VMEM((BPW,), jnp.int32),
            pltpu.VMEM((BPW,), jnp.int32),
            pltpu.VMEM((BPW,), jnp.int32),
            pltpu.VMEM((L, 2 * FACTORS), jnp.float32),
            pltpu.VMEM((L, 2 * FACTORS), jnp.float32),
            pltpu.VMEM((L, 2 * FACTORS), jnp.float32),
            pltpu.VMEM((L, L), jnp.float32),
            pltpu.VMEM((BPW,), jnp.float32),
            pltpu.SemaphoreType.DMA,
        ],
    )
    uf2 = user_factors.reshape(-1, 2 * FACTORS)
    if2 = item_factors.reshape(-1, 2 * FACTORS)
    return gather(user2, ii2, ij2, uf2, if2)
